# Initial kernel scaffold; baseline (speedup 1.0000x reference)
import functools

import jax
import jax.numpy as jnp
from jax import lax
from jax.experimental import pallas as pl
from jax.experimental.pallas import tpu as pltpu
from jax.experimental.pallas import tpu_sc as plsc

N = 10000
E0 = 60000
NT = 32
PT = 1920
EH = NT * PT
E = 2 * EH
CH = 128
NCH = PT // CH
TBL = 100_000_008
PADKEY = 100_000_000
DUMP = N
TR = 10240
SPT = 2 * PT
SNC = SPT // CH
RB = 512
NEB = E // RB
HEB = NEB // 2
NB = 1000

_mesh = plsc.VectorSubcoreMesh(core_axis_name="c", subcore_axis_name="s")
f32 = jnp.float32
i32 = jnp.int32


def _wid():
    return lax.axis_index("s") * 2 + lax.axis_index("c")


@functools.partial(
    pl.kernel,
    out_type=jax.ShapeDtypeStruct((TBL,), i32),
    mesh=_mesh,
    scratch_types=[
        pltpu.VMEM((NCH, CH), i32),
        pltpu.VMEM((PT,), i32),
        pltpu.SemaphoreType.DMA,
    ],
)
def _dedup_scatter(key_hbm, tbl_hbm, key_v, val_v, sem):
    wid = _wid()
    base = wid * PT
    pltpu.sync_copy(key_hbm.at[wid], key_v)

    def fill(t, c):
        val_v[pl.ds(t * 16, 16)] = lax.iota(i32, 16) + (base + t * 16)
        return c

    lax.fori_loop(0, PT // 16, fill, 0)

    def scat(j, c):
        pltpu.async_copy(
            val_v.at[pl.ds(j * CH, CH)], tbl_hbm.at[key_v.at[j]], sem
        ).wait()
        return c

    lax.fori_loop(0, NCH, scat, 0)


@functools.partial(
    pl.kernel,
    out_type=[
        jax.ShapeDtypeStruct((EH * 8,), f32),
        jax.ShapeDtypeStruct((EH,), i32),
        jax.ShapeDtypeStruct((EH,), i32),
    ],
    mesh=_mesh,
    scratch_types=[
        pltpu.VMEM((NCH, CH), i32),
        pltpu.VMEM((PT,), i32),
        pltpu.VMEM((PT,), i32),
        pltpu.VMEM((PT,), i32),
        pltpu.VMEM((N,), f32),
        pltpu.VMEM((N,), f32),
        pltpu.VMEM((PT * 8,), f32),
        pltpu.VMEM((PT,), i32),
        pltpu.VMEM((PT,), i32),
        pltpu.SemaphoreType.DMA,
    ],
)
def _dedup_features(key_hbm, lo_hbm, hi_hbm, mx_hbm, my_hbm, zf_hbm, tbl_hbm,
                    feat_hbm, agga_hbm, aggb_hbm,
                    key_v, lo_v, hi_v, w_v, mx_v, my_v, feat_v, aa_v, ab_v, sem):
    wid = _wid()
    base = wid * PT
    pltpu.sync_copy(key_hbm.at[wid], key_v)
    pltpu.sync_copy(lo_hbm.at[wid], lo_v)
    pltpu.sync_copy(hi_hbm.at[wid], hi_v)
    pltpu.sync_copy(mx_hbm, mx_v)
    pltpu.sync_copy(my_hbm, my_v)
    pltpu.sync_copy(zf_hbm, feat_v)

    def gat(j, c):
        pltpu.async_copy(
            tbl_hbm.at[key_v.at[j]], w_v.at[pl.ds(j * CH, CH)], sem
        ).wait()
        return c

    lax.fori_loop(0, NCH, gat, 0)

    def body(t, c):
        sl = pl.ds(t * 16, 16)
        lo16 = lo_v[sl]
        hi16 = hi_v[sl]
        w16 = w_v[sl]
        g16 = lax.iota(i32, 16) + (base + t * 16)
        rep = (w16 == g16) & (g16 < E0)
        ax = plsc.load_gather(mx_v, [lo16]) - plsc.load_gather(mx_v, [hi16])
        ay = plsc.load_gather(my_v, [lo16]) - plsc.load_gather(my_v, [hi16])
        n2 = ax * ax + ay * ay
        y = plsc.bitcast(0x5F3759DF - (plsc.bitcast(n2, i32) >> 1), f32)
        y = y * (1.5 - 0.5 * n2 * y * y)
        y = y * (1.5 - 0.5 * n2 * y * y)
        y = y * (1.5 - 0.5 * n2 * y * y)
        nrm = jnp.where(n2 > 0.0, n2 * y, 0.0)
        repf = jnp.where(rep, 1.0, 0.0).astype(f32)
        fb = (lax.iota(i32, 16) + t * 16) * 8
        plsc.store_scatter(feat_v, [fb], ax)
        plsc.store_scatter(feat_v, [fb + 1], ay)
        plsc.store_scatter(feat_v, [fb + 2], nrm)
        plsc.store_scatter(feat_v, [fb + 3], repf)
        aa_v[sl] = jnp.where(rep, hi16, DUMP)
        ab_v[sl] = jnp.where(rep, lo16, DUMP)
        return c

    lax.fori_loop(0, PT // 16, body, 0)
    pltpu.sync_copy(feat_v, feat_hbm.at[pl.ds(base * 8, PT * 8)])
    pltpu.sync_copy(aa_v, agga_hbm.at[pl.ds(base, PT)])
    pltpu.sync_copy(ab_v, aggb_hbm.at[pl.ds(base, PT)])


@functools.partial(
    pl.kernel,
    out_type=[
        jax.ShapeDtypeStruct((EH, 128), f32),
        jax.ShapeDtypeStruct((EH, 128), f32),
    ],
    mesh=_mesh,
    scratch_types=[
        pltpu.VMEM((NCH, CH), i32),
        pltpu.VMEM((NCH, CH), i32),
        pltpu.VMEM((CH, 128), f32),
        pltpu.VMEM((CH, 128), f32),
        pltpu.SemaphoreType.DMA,
        pltpu.SemaphoreType.DMA,
    ],
)
def _gather_nodes(nodes_hbm, lohi_hbm, gl_hbm, gh_hbm,
                  il_v, ih_v, bl, bh, sem1, sem2):
    wid = _wid()
    base = wid * PT
    pltpu.sync_copy(lohi_hbm.at[0, wid], il_v)
    pltpu.sync_copy(lohi_hbm.at[1, wid], ih_v)

    def body(j, c):
        pltpu.async_copy(nodes_hbm.at[il_v.at[j]], bl, sem1).wait()
        pltpu.sync_copy(bl, gl_hbm.at[pl.ds(base + j * CH, CH)])
        pltpu.async_copy(nodes_hbm.at[ih_v.at[j]], bh, sem2).wait()
        pltpu.sync_copy(bh, gh_hbm.at[pl.ds(base + j * CH, CH)])
        return c

    lax.fori_loop(0, NCH, body, 0)


@functools.partial(
    pl.kernel,
    out_type=jax.ShapeDtypeStruct((2 * N, 128), f32),
    mesh=_mesh,
    scratch_types=[
        pltpu.VMEM((SNC, CH), i32),
        pltpu.VMEM((CH, 128), f32),
        pltpu.VMEM_SHARED((TR, 128), f32),
        pltpu.SemaphoreType.DMA,
    ],
)
def _segment_sum(edges_hbm, agg_hbm, zer_hbm, p_hbm, idx_v, ebuf, tbl_s, sem):
    cid = lax.axis_index("c")
    sid = lax.axis_index("s")
    pltpu.sync_copy(agg_hbm.at[cid, sid], idx_v)
    pltpu.sync_copy(zer_hbm.at[pl.ds(sid * (TR // 16), TR // 16)],
                    tbl_s.at[pl.ds(sid * (TR // 16), TR // 16)])
    plsc.subcore_barrier()
    base = cid * EH + sid * SPT

    def body(j, c):
        pltpu.async_copy(edges_hbm.at[pl.ds(base + j * CH, CH)], ebuf, sem).wait()
        pltpu.sync_copy(ebuf, tbl_s.at[idx_v.at[j]], add=True)
        return c

    lax.fori_loop(0, SNC, body, 0)
    plsc.subcore_barrier()
    rpt = N // 16
    pltpu.sync_copy(tbl_s.at[pl.ds(sid * rpt, rpt)],
                    p_hbm.at[pl.ds(cid * N + sid * rpt, rpt)])


def _ln(h, g, bb):
    mu = jnp.mean(h, axis=-1, keepdims=True)
    d = h - mu
    var = jnp.mean(d * d, axis=-1, keepdims=True)
    return d * lax.rsqrt(var + 1e-5) * g + bb


def _node_stats(nf16):
    def body(x_ref, mean_ref, std_ref):
        x = x_ref[...]
        mu = jnp.mean(x, axis=0, keepdims=True)
        ex2 = jnp.mean(x * x, axis=0, keepdims=True)
        sd = jnp.sqrt(jnp.maximum(ex2 - mu * mu, 0.0))
        mean_ref[...] = mu
        std_ref[...] = jnp.maximum(sd, 1e-8)

    return pl.pallas_call(
        body,
        out_shape=[jax.ShapeDtypeStruct((1, 16), f32),
                   jax.ShapeDtypeStruct((1, 16), f32)],
    )(nf16)


def _edge_stats(feat):
    def body(f_ref, mean_ref, std_ref):
        x = f_ref[...]
        rx = x[:, 0:1]
        ry = x[:, 1:2]
        nm = x[:, 2:3]
        rp = x[:, 3:4]
        cnt = jnp.sum(rp)
        sn = jnp.sum(nm * rp) / cnt
        ex2x = jnp.sum(rx * rx * rp) / cnt
        ex2y = jnp.sum(ry * ry * rp) / cnt
        ex2n = jnp.sum(nm * nm * rp) / cnt
        sdx = jnp.maximum(jnp.sqrt(jnp.maximum(ex2x, 0.0)), 1e-8)
        sdy = jnp.maximum(jnp.sqrt(jnp.maximum(ex2y, 0.0)), 1e-8)
        sdn = jnp.maximum(jnp.sqrt(jnp.maximum(ex2n - sn * sn, 0.0)), 1e-8)
        col = lax.broadcasted_iota(i32, (1, 8), 1)
        mean_ref[...] = jnp.where(col == 2, sn, 0.0)
        std_ref[...] = jnp.where(
            col == 0, sdx, jnp.where(col == 1, sdy, jnp.where(col == 2, sdn, 1.0)))

    return pl.pallas_call(
        body,
        out_shape=[jax.ShapeDtypeStruct((1, 8), f32),
                   jax.ShapeDtypeStruct((1, 8), f32)],
    )(feat)


def _mm(a, b):
    return jnp.dot(a, b, preferred_element_type=f32)


def _node_encoder(nf16, mean, std, w1, b1, w2, b2, w3, b3, g, bb):
    def body(x_ref, m_ref, s_ref, w1r, b1r, w2r, b2r, w3r, b3r, gr, bbr, o_ref):
        x = (x_ref[...] - m_ref[...]) / s_ref[...]
        h = jnp.maximum(_mm(x, w1r[...]) + b1r[...], 0.0)
        h = jnp.maximum(_mm(h, w2r[...]) + b2r[...], 0.0)
        h = _mm(h, w3r[...]) + b3r[...]
        o_ref[...] = _ln(h, gr[...], bbr[...])

    z = lambda i: (0, 0)
    return pl.pallas_call(
        body,
        grid=(N // NB,),
        in_specs=[
            pl.BlockSpec((NB, 16), lambda i: (i, 0)),
            pl.BlockSpec((1, 16), z), pl.BlockSpec((1, 16), z),
            pl.BlockSpec((16, 128), z), pl.BlockSpec((1, 128), z),
            pl.BlockSpec((128, 128), z), pl.BlockSpec((1, 128), z),
            pl.BlockSpec((128, 128), z), pl.BlockSpec((1, 128), z),
            pl.BlockSpec((1, 128), z), pl.BlockSpec((1, 128), z),
        ],
        out_specs=pl.BlockSpec((NB, 128), lambda i: (i, 0)),
        out_shape=jax.ShapeDtypeStruct((N, 128), f32),
    )(nf16, mean, std, w1, b1, w2, b2, w3, b3, g, bb)


def _edge_encoder(feat, mean, std, w1, b1, w2, b2, w3, b3, g, bb):
    def body(f_ref, m_ref, s_ref, w1r, b1r, w2r, b2r, w3r, b3r, gr, bbr, o_ref):
        i = pl.program_id(0)
        sgn = jnp.where(i < HEB, 1.0, -1.0)
        col = lax.broadcasted_iota(i32, (1, 8), 1)
        sv = jnp.where(col < 2, sgn, 1.0)
        x = (f_ref[...] - m_ref[...]) / s_ref[...] * sv
        h = jnp.maximum(_mm(x, w1r[...]) + b1r[...], 0.0)
        h = jnp.maximum(_mm(h, w2r[...]) + b2r[...], 0.0)
        h = _mm(h, w3r[...]) + b3r[...]
        o_ref[...] = _ln(h, gr[...], bbr[...])

    z = lambda i: (0, 0)
    return pl.pallas_call(
        body,
        grid=(NEB,),
        in_specs=[
            pl.BlockSpec((RB, 8), lambda i: (i % HEB, 0)),
            pl.BlockSpec((1, 8), z), pl.BlockSpec((1, 8), z),
            pl.BlockSpec((8, 128), z), pl.BlockSpec((1, 128), z),
            pl.BlockSpec((128, 128), z), pl.BlockSpec((1, 128), z),
            pl.BlockSpec((128, 128), z), pl.BlockSpec((1, 128), z),
            pl.BlockSpec((1, 128), z), pl.BlockSpec((1, 128), z),
        ],
        out_specs=pl.BlockSpec((RB, 128), lambda i: (i, 0)),
        out_shape=jax.ShapeDtypeStruct((E, 128), f32),
    )(feat, mean, std, w1, b1, w2, b2, w3, b3, g, bb)


def _edge_mlp(edges, gl, gh, we, ws, wr, b1, w2, b2, w3, b3, g, bb):
    def body(e_ref, l_ref, h_ref, wer, wsr, wrr, b1r, w2r, b2r, w3r, b3r,
             gr, bbr, o_ref):
        i = pl.program_id(0)
        first = i < HEB
        lv = l_ref[...]
        hv = h_ref[...]
        s = jnp.where(first, lv, hv)
        r = jnp.where(first, hv, lv)
        e = e_ref[...]
        h = _mm(e, wer[...]) + _mm(s, wsr[...]) + _mm(r, wrr[...]) + b1r[...]
        h = jnp.maximum(h, 0.0)
        h = jnp.maximum(_mm(h, w2r[...]) + b2r[...], 0.0)
        h = _mm(h, w3r[...]) + b3r[...]
        o_ref[...] = e + _ln(h, gr[...], bbr[...])

    z = lambda i: (0, 0)
    return pl.pallas_call(
        body,
        grid=(NEB,),
        in_specs=[
            pl.BlockSpec((RB, 128), lambda i: (i, 0)),
            pl.BlockSpec((RB, 128), lambda i: (i % HEB, 0)),
            pl.BlockSpec((RB, 128), lambda i: (i % HEB, 0)),
            pl.BlockSpec((128, 128), z), pl.BlockSpec((128, 128), z),
            pl.BlockSpec((128, 128), z), pl.BlockSpec((1, 128), z),
            pl.BlockSpec((128, 128), z), pl.BlockSpec((1, 128), z),
            pl.BlockSpec((128, 128), z), pl.BlockSpec((1, 128), z),
            pl.BlockSpec((1, 128), z), pl.BlockSpec((1, 128), z),
        ],
        out_specs=pl.BlockSpec((RB, 128), lambda i: (i, 0)),
        out_shape=jax.ShapeDtypeStruct((E, 128), f32),
    )(edges, gl, gh, we, ws, wr, b1, w2, b2, w3, b3, g, bb)


def _node_mlp(nodes, p, wn, wa, b1, w2, b2, w3, b3, g, bb):
    def body(n_ref, p0_ref, p1_ref, wnr, war, b1r, w2r, b2r, w3r, b3r,
             gr, bbr, o_ref):
        nd = n_ref[...]
        agg = p0_ref[...] + p1_ref[...]
        h = _mm(nd, wnr[...]) + _mm(agg, war[...]) + b1r[...]
        h = jnp.maximum(h, 0.0)
        h = jnp.maximum(_mm(h, w2r[...]) + b2r[...], 0.0)
        h = _mm(h, w3r[...]) + b3r[...]
        o_ref[...] = nd + _ln(h, gr[...], bbr[...])

    z = lambda i: (0, 0)
    return pl.pallas_call(
        body,
        grid=(N // NB,),
        in_specs=[
            pl.BlockSpec((NB, 128), lambda i: (i, 0)),
            pl.BlockSpec((NB, 128), lambda i: (i, 0)),
            pl.BlockSpec((NB, 128), lambda i: (i + N // NB, 0)),
            pl.BlockSpec((128, 128), z), pl.BlockSpec((128, 128), z),
            pl.BlockSpec((1, 128), z),
            pl.BlockSpec((128, 128), z), pl.BlockSpec((1, 128), z),
            pl.BlockSpec((128, 128), z), pl.BlockSpec((1, 128), z),
            pl.BlockSpec((1, 128), z), pl.BlockSpec((1, 128), z),
        ],
        out_specs=pl.BlockSpec((NB, 128), lambda i: (i, 0)),
        out_shape=jax.ShapeDtypeStruct((N, 128), f32),
    )(nodes, p, p, wn, wa, b1, w2, b2, w3, b3, g, bb)


def _decoder(nodes, w1, b1, w2p, b2p):
    def body(n_ref, w1r, b1r, w2r, b2r, o_ref):
        h = jnp.maximum(_mm(n_ref[...], w1r[...]) + b1r[...], 0.0)
        o_ref[...] = _mm(h, w2r[...]) + b2r[...]

    z = lambda i: (0, 0)
    return pl.pallas_call(
        body,
        grid=(N // NB,),
        in_specs=[
            pl.BlockSpec((NB, 128), lambda i: (i, 0)),
            pl.BlockSpec((128, 128), z), pl.BlockSpec((1, 128), z),
            pl.BlockSpec((128, 128), z), pl.BlockSpec((1, 128), z),
        ],
        out_specs=pl.BlockSpec((NB, 128), lambda i: (i, 0)),
        out_shape=jax.ShapeDtypeStruct((N, 128), f32),
    )(nodes, w1, b1, w2p, b2p)


def _row(b):
    return b.reshape(1, -1)


def kernel(velocity, mesh_pos, node_type, cells, is_training, params):
    del is_training
    c = cells.astype(i32)
    ea = jnp.concatenate([c[:, 0], c[:, 1], c[:, 2]])
    eb = jnp.concatenate([c[:, 1], c[:, 2], c[:, 0]])
    lo = jnp.minimum(ea, eb)
    hi = jnp.maximum(ea, eb)
    pad = EH - E0
    lo_p = jnp.concatenate([lo, jnp.zeros((pad,), i32)])
    hi_p = jnp.concatenate([hi, jnp.zeros((pad,), i32)])
    key_p = jnp.concatenate([lo * N + hi, jnp.full((pad,), PADKEY, i32)])
    key3 = key_p.reshape(NT, NCH, CH)
    lo2 = lo_p.reshape(NT, PT)
    hi2 = hi_p.reshape(NT, PT)
    mx = jnp.ascontiguousarray(mesh_pos[:, 0])
    my = jnp.ascontiguousarray(mesh_pos[:, 1])
    zflat = jnp.zeros((PT * 8,), f32)

    tbl = _dedup_scatter(key3)
    featf, agga, aggb = _dedup_features(key3, lo2, hi2, mx, my, zflat, tbl)
    feat = featf.reshape(EH, 8)
    aggd = jnp.stack([agga.reshape(16, SNC, CH), aggb.reshape(16, SNC, CH)])
    lohi = jnp.stack([lo2.reshape(NT, NCH, CH), hi2.reshape(NT, NCH, CH)])
    zer = jnp.zeros((TR, 128), f32)

    nt1h = jax.nn.one_hot(node_type[:, 0], 9, dtype=f32)
    nf16 = jnp.concatenate([velocity, nt1h, jnp.zeros((N, 5), f32)], axis=1)

    def mlp3(p):
        (w1, b1), (w2, b2), (w3, b3) = p
        return w1, _row(b1), w2, _row(b2), w3, _row(b3)

    nw1, nb1, nw2, nb2, nw3, nb3 = mlp3(params['enc_node']['mlp'])
    nw1p = jnp.zeros((16, 128), f32).at[:11].set(nw1)
    ng, nbb = params['enc_node']['ln']
    nmean, nstd = _node_stats(nf16)
    nodes = _node_encoder(nf16, nmean, nstd, nw1p, nb1, nw2, nb2, nw3, nb3,
                          _row(ng), _row(nbb))

    ew1, eb1, ew2, eb2, ew3, eb3 = mlp3(params['enc_edge']['mlp'])
    ew1p = jnp.zeros((8, 128), f32).at[:3].set(ew1)
    eg, ebb = params['enc_edge']['ln']
    emean, estd = _edge_stats(feat)
    edges = _edge_encoder(feat, emean, estd, ew1p, eb1, ew2, eb2, ew3, eb3,
                          _row(eg), _row(ebb))

    for blk in params['blocks']:
        (w1, b1), (w2, b2), (w3, b3) = blk['edge_mlp']
        we, wsnd, wrcv = w1[:128], w1[128:256], w1[256:]
        eg_, ebb_ = blk['edge_ln']
        (nw1_, nb1_), (nw2_, nb2_), (nw3_, nb3_) = blk['node_mlp']
        wn, wa = nw1_[:128], nw1_[128:]
        ng_, nbb_ = blk['node_ln']

        gl, gh = _gather_nodes(nodes, lohi)
        edges = _edge_mlp(edges, gl, gh, we, wsnd, wrcv, _row(b1),
                          w2, _row(b2), w3, _row(b3), _row(eg_), _row(ebb_))
        p = _segment_sum(edges, aggd, zer)
        nodes = _node_mlp(nodes, p, wn, wa, _row(nb1_),
                          nw2_, _row(nb2_), nw3_, _row(nb3_),
                          _row(ng_), _row(nbb_))

    (dw1, db1), (dw2, db2) = params['dec']['mlp']
    dw2p = jnp.zeros((128, 128), f32).at[:, :2].set(dw2)
    db2p = jnp.zeros((1, 128), f32).at[0, :2].set(db2)
    out = _decoder(nodes, dw1, _row(db1), dw2p, db2p)
    return out[:, :2]


# speedup vs baseline: 2.0757x; 2.0757x over previous
import functools

import jax
import jax.numpy as jnp
from jax import lax
from jax.experimental import pallas as pl
from jax.experimental.pallas import tpu as pltpu
from jax.experimental.pallas import tpu_sc as plsc

N = 10000
E0 = 60000
NT = 32
PT = 1920
EH = NT * PT
E = 2 * EH
CH = 128
NCH = PT // CH
TBL = 100_000_008
PADKEY = 100_000_000
DUMP = N
TR = 10240
SPT = 2 * PT
SNC = SPT // CH
RB = 512
NEB = E // RB
HEB = NEB // 2
NB = 1000

_mesh = plsc.VectorSubcoreMesh(core_axis_name="c", subcore_axis_name="s",
                               num_cores=2, num_subcores=16)
_sc_params = pltpu.CompilerParams(needs_layout_passes=False)
f32 = jnp.float32
i32 = jnp.int32


def _wid():
    return lax.axis_index("s") * 2 + lax.axis_index("c")


@functools.partial(
    pl.kernel,
    out_type=jax.ShapeDtypeStruct((TBL,), i32),
    mesh=_mesh,
    compiler_params=_sc_params,
    scratch_types=[
        pltpu.VMEM((NCH, CH), i32),
        pltpu.VMEM((PT,), i32),
        pltpu.SemaphoreType.DMA,
    ],
)
def _dedup_scatter(key_hbm, tbl_hbm, key_v, val_v, sem):
    wid = _wid()
    base = wid * PT
    pltpu.sync_copy(key_hbm.at[wid], key_v)

    def fill(t, c):
        val_v[pl.ds(t * 16, 16)] = lax.iota(i32, 16) + (base + t * 16)
        return c

    lax.fori_loop(0, PT // 16, fill, 0)

    def scat(j, c):
        pltpu.async_copy(
            val_v.at[pl.ds(j * CH, CH)], tbl_hbm.at[key_v.at[j]], sem
        ).wait()
        return c

    lax.fori_loop(0, NCH, scat, 0)


@functools.partial(
    pl.kernel,
    out_type=[
        jax.ShapeDtypeStruct((EH * 8,), f32),
        jax.ShapeDtypeStruct((EH,), i32),
        jax.ShapeDtypeStruct((EH,), i32),
    ],
    mesh=_mesh,
    compiler_params=_sc_params,
    scratch_types=[
        pltpu.VMEM((NCH, CH), i32),
        pltpu.VMEM((PT,), i32),
        pltpu.VMEM((PT,), i32),
        pltpu.VMEM((PT,), i32),
        pltpu.VMEM((N,), f32),
        pltpu.VMEM((N,), f32),
        pltpu.VMEM((PT * 8,), f32),
        pltpu.VMEM((PT,), i32),
        pltpu.VMEM((PT,), i32),
        pltpu.SemaphoreType.DMA,
    ],
)
def _dedup_features(key_hbm, lo_hbm, hi_hbm, mx_hbm, my_hbm, zf_hbm, tbl_hbm,
                    feat_hbm, agga_hbm, aggb_hbm,
                    key_v, lo_v, hi_v, w_v, mx_v, my_v, feat_v, aa_v, ab_v, sem):
    wid = _wid()
    base = wid * PT
    pltpu.sync_copy(key_hbm.at[wid], key_v)
    pltpu.sync_copy(lo_hbm.at[wid], lo_v)
    pltpu.sync_copy(hi_hbm.at[wid], hi_v)
    pltpu.sync_copy(mx_hbm, mx_v)
    pltpu.sync_copy(my_hbm, my_v)
    pltpu.sync_copy(zf_hbm, feat_v)

    def gat(j, c):
        pltpu.async_copy(
            tbl_hbm.at[key_v.at[j]], w_v.at[pl.ds(j * CH, CH)], sem
        ).wait()
        return c

    lax.fori_loop(0, NCH, gat, 0)

    def body(t, c):
        sl = pl.ds(t * 16, 16)
        lo16 = lo_v[sl]
        hi16 = hi_v[sl]
        w16 = w_v[sl]
        g16 = lax.iota(i32, 16) + (base + t * 16)
        rep = (w16 == g16) & (g16 < E0)
        ax = plsc.load_gather(mx_v, [lo16]) - plsc.load_gather(mx_v, [hi16])
        ay = plsc.load_gather(my_v, [lo16]) - plsc.load_gather(my_v, [hi16])
        n2 = ax * ax + ay * ay
        repf = jnp.where(rep, 1.0, 0.0).astype(f32)
        fb = (lax.iota(i32, 16) + t * 16) * 8
        plsc.store_scatter(feat_v, [fb], ax)
        plsc.store_scatter(feat_v, [fb + 1], ay)
        plsc.store_scatter(feat_v, [fb + 2], n2)
        plsc.store_scatter(feat_v, [fb + 3], repf)
        aa_v[sl] = jnp.where(rep, hi16, DUMP)
        ab_v[sl] = jnp.where(rep, lo16, DUMP)
        return c

    lax.fori_loop(0, PT // 16, body, 0)
    pltpu.sync_copy(feat_v, feat_hbm.at[pl.ds(base * 8, PT * 8)])
    pltpu.sync_copy(aa_v, agga_hbm.at[pl.ds(base, PT)])
    pltpu.sync_copy(ab_v, aggb_hbm.at[pl.ds(base, PT)])


@functools.partial(
    pl.kernel,
    out_type=[
        jax.ShapeDtypeStruct((EH, 128), f32),
        jax.ShapeDtypeStruct((EH, 128), f32),
    ],
    mesh=_mesh,
    compiler_params=_sc_params,
    scratch_types=[
        pltpu.VMEM((NCH, CH), i32),
        pltpu.VMEM((NCH, CH), i32),
        pltpu.VMEM((CH, 128), f32),
        pltpu.VMEM((CH, 128), f32),
        pltpu.SemaphoreType.DMA,
        pltpu.SemaphoreType.DMA,
    ],
)
def _gather_nodes(nodes_hbm, lohi_hbm, gl_hbm, gh_hbm,
                  il_v, ih_v, bl, bh, sem1, sem2):
    wid = _wid()
    base = wid * PT
    pltpu.sync_copy(lohi_hbm.at[0, wid], il_v)
    pltpu.sync_copy(lohi_hbm.at[1, wid], ih_v)

    def body(j, c):
        pltpu.async_copy(nodes_hbm.at[il_v.at[j]], bl, sem1).wait()
        pltpu.sync_copy(bl, gl_hbm.at[pl.ds(base + j * CH, CH)])
        pltpu.async_copy(nodes_hbm.at[ih_v.at[j]], bh, sem2).wait()
        pltpu.sync_copy(bh, gh_hbm.at[pl.ds(base + j * CH, CH)])
        return c

    lax.fori_loop(0, NCH, body, 0)


@functools.partial(
    pl.kernel,
    out_type=jax.ShapeDtypeStruct((2, TR, 128), f32),
    mesh=_mesh,
    compiler_params=_sc_params,
    scratch_types=[
        pltpu.VMEM((SNC, CH), i32),
        pltpu.VMEM((CH, 128), f32),
        pltpu.VMEM_SHARED((TR, 128), f32),
        pltpu.SemaphoreType.DMA,
    ],
)
def _segment_sum(edges_hbm, agg_hbm, zer_hbm, p_hbm, idx_v, ebuf, tbl_s, sem):
    cid = lax.axis_index("c")
    sid = lax.axis_index("s")
    pltpu.sync_copy(agg_hbm.at[cid, sid], idx_v)
    pltpu.sync_copy(zer_hbm.at[pl.ds(sid * (TR // 16), TR // 16)],
                    tbl_s.at[pl.ds(sid * (TR // 16), TR // 16)])
    plsc.subcore_barrier()
    base = cid * EH + sid * SPT

    def body(j, c):
        pltpu.async_copy(edges_hbm.at[pl.ds(base + j * CH, CH)], ebuf, sem).wait()
        pltpu.sync_copy(ebuf, tbl_s.at[idx_v.at[j]], add=True)
        return c

    lax.fori_loop(0, SNC, body, 0)
    plsc.subcore_barrier()
    rpt = TR // 16
    pltpu.sync_copy(tbl_s.at[pl.ds(sid * rpt, rpt)],
                    p_hbm.at[cid, pl.ds(sid * rpt, rpt)])


def _ln(h, g, bb):
    mu = jnp.mean(h, axis=-1, keepdims=True)
    d = h - mu
    var = jnp.mean(d * d, axis=-1, keepdims=True)
    return d / jnp.sqrt(var + 1e-5) * g + bb


def _node_stats(nf16):
    def body(x_ref, mean_ref, std_ref):
        x = x_ref[...]
        mu = jnp.mean(x, axis=0, keepdims=True)
        ex2 = jnp.mean(x * x, axis=0, keepdims=True)
        sd = jnp.sqrt(jnp.maximum(ex2 - mu * mu, 0.0))
        mean_ref[...] = mu
        std_ref[...] = jnp.maximum(sd, 1e-8)

    return pl.pallas_call(
        body,
        out_shape=[jax.ShapeDtypeStruct((1, 16), f32),
                   jax.ShapeDtypeStruct((1, 16), f32)],
    )(nf16)


def _edge_sums(feat):
    def body(f_ref, o_ref):
        i = pl.program_id(0)
        x = f_ref[...]
        rx = x[:, 0:1]
        ry = x[:, 1:2]
        n2 = x[:, 2:3]
        nm = jnp.sqrt(n2)
        rp = x[:, 3:4]
        s0 = jnp.sum(rp)
        s1 = jnp.sum(nm * rp)
        s2 = jnp.sum(rx * rx * rp)
        s3 = jnp.sum(ry * ry * rp)
        s4 = jnp.sum(n2 * rp)
        col = lax.broadcasted_iota(i32, (1, 8), 1)
        vals = jnp.where(
            col == 0, s0,
            jnp.where(col == 1, s1,
                      jnp.where(col == 2, s2,
                                jnp.where(col == 3, s3,
                                          jnp.where(col == 4, s4, 0.0)))))
        o_ref[...] = jnp.where(i == 0, vals, o_ref[...] + vals)

    return pl.pallas_call(
        body,
        grid=(EH // RB,),
        in_specs=[pl.BlockSpec((RB, 8), lambda i: (i, 0))],
        out_specs=pl.BlockSpec((1, 8), lambda i: (0, 0)),
        out_shape=jax.ShapeDtypeStruct((1, 8), f32),
    )(feat)


def _mm(a, b):
    return jnp.dot(a, b, preferred_element_type=f32)


def _node_encoder(nf16, mean, std, w1, b1, w2, b2, w3, b3, g, bb):
    def body(x_ref, m_ref, s_ref, w1r, b1r, w2r, b2r, w3r, b3r, gr, bbr, o_ref):
        x = (x_ref[...] - m_ref[...]) / s_ref[...]
        h = jnp.maximum(_mm(x, w1r[...]) + b1r[...], 0.0)
        h = jnp.maximum(_mm(h, w2r[...]) + b2r[...], 0.0)
        h = _mm(h, w3r[...]) + b3r[...]
        o_ref[...] = _ln(h, gr[...], bbr[...])

    z = lambda i: (0, 0)
    return pl.pallas_call(
        body,
        grid=(N // NB,),
        in_specs=[
            pl.BlockSpec((NB, 16), lambda i: (i, 0)),
            pl.BlockSpec((1, 16), z), pl.BlockSpec((1, 16), z),
            pl.BlockSpec((16, 128), z), pl.BlockSpec((1, 128), z),
            pl.BlockSpec((128, 128), z), pl.BlockSpec((1, 128), z),
            pl.BlockSpec((128, 128), z), pl.BlockSpec((1, 128), z),
            pl.BlockSpec((1, 128), z), pl.BlockSpec((1, 128), z),
        ],
        out_specs=pl.BlockSpec((NB, 128), lambda i: (i, 0)),
        out_shape=jax.ShapeDtypeStruct((N, 128), f32),
    )(nf16, mean, std, w1, b1, w2, b2, w3, b3, g, bb)


def _edge_encoder(feat, sums, w1, b1, w2, b2, w3, b3, g, bb):
    def body(f_ref, s_ref, w1r, b1r, w2r, b2r, w3r, b3r, gr, bbr, o_ref):
        i = pl.program_id(0)
        cnt = s_ref[0, 0]
        sn = s_ref[0, 1] / cnt
        sdx = jnp.maximum(jnp.sqrt(jnp.maximum(s_ref[0, 2] / cnt, 0.0)), 1e-8)
        sdy = jnp.maximum(jnp.sqrt(jnp.maximum(s_ref[0, 3] / cnt, 0.0)), 1e-8)
        sdn = jnp.maximum(
            jnp.sqrt(jnp.maximum(s_ref[0, 4] / cnt - sn * sn, 0.0)), 1e-8)
        col = lax.broadcasted_iota(i32, (1, 8), 1)
        mean = jnp.where(col == 2, sn, 0.0)
        std = jnp.where(
            col == 0, sdx, jnp.where(col == 1, sdy, jnp.where(col == 2, sdn, 1.0)))
        sgn = jnp.where(i < HEB, 1.0, -1.0)
        sv = jnp.where(col < 2, sgn, 1.0)
        f = f_ref[...]
        f = jnp.where(col == 2, jnp.sqrt(jnp.maximum(f, 0.0)), f)
        x = (f - mean) / std * sv
        h = jnp.maximum(_mm(x, w1r[...]) + b1r[...], 0.0)
        h = jnp.maximum(_mm(h, w2r[...]) + b2r[...], 0.0)
        h = _mm(h, w3r[...]) + b3r[...]
        o_ref[...] = _ln(h, gr[...], bbr[...])

    z = lambda i: (0, 0)
    return pl.pallas_call(
        body,
        grid=(NEB,),
        in_specs=[
            pl.BlockSpec((RB, 8), lambda i: (i % HEB, 0)),
            pl.BlockSpec((1, 8), z),
            pl.BlockSpec((8, 128), z), pl.BlockSpec((1, 128), z),
            pl.BlockSpec((128, 128), z), pl.BlockSpec((1, 128), z),
            pl.BlockSpec((128, 128), z), pl.BlockSpec((1, 128), z),
            pl.BlockSpec((1, 128), z), pl.BlockSpec((1, 128), z),
        ],
        out_specs=pl.BlockSpec((RB, 128), lambda i: (i, 0)),
        out_shape=jax.ShapeDtypeStruct((E, 128), f32),
    )(feat, sums, w1, b1, w2, b2, w3, b3, g, bb)


def _edge_mlp(edges, gl, gh, w1, b1, w2, b2, w3, b3, g, bb):
    def body(e_ref, l_ref, h_ref, w1r, b1r, w2r, b2r, w3r, b3r,
             gr, bbr, o_ref):
        i = pl.program_id(0)
        first = i < HEB
        lv = l_ref[...]
        hv = h_ref[...]
        s = jnp.where(first, lv, hv)
        r = jnp.where(first, hv, lv)
        e = e_ref[...]
        h = _mm(jnp.concatenate([e, s, r], axis=1), w1r[...]) + b1r[...]
        h = jnp.maximum(h, 0.0)
        h = jnp.maximum(_mm(h, w2r[...]) + b2r[...], 0.0)
        h = _mm(h, w3r[...]) + b3r[...]
        o_ref[...] = e + _ln(h, gr[...], bbr[...])

    z = lambda i: (0, 0)
    return pl.pallas_call(
        body,
        grid=(NEB,),
        in_specs=[
            pl.BlockSpec((RB, 128), lambda i: (i, 0)),
            pl.BlockSpec((RB, 128), lambda i: (i % HEB, 0)),
            pl.BlockSpec((RB, 128), lambda i: (i % HEB, 0)),
            pl.BlockSpec((384, 128), z), pl.BlockSpec((1, 128), z),
            pl.BlockSpec((128, 128), z), pl.BlockSpec((1, 128), z),
            pl.BlockSpec((128, 128), z), pl.BlockSpec((1, 128), z),
            pl.BlockSpec((1, 128), z), pl.BlockSpec((1, 128), z),
        ],
        out_specs=pl.BlockSpec((RB, 128), lambda i: (i, 0)),
        out_shape=jax.ShapeDtypeStruct((E, 128), f32),
    )(edges, gl, gh, w1, b1, w2, b2, w3, b3, g, bb)


def _node_mlp(nodes, p, w1, b1, w2, b2, w3, b3, g, bb):
    def body(n_ref, p0_ref, p1_ref, w1r, b1r, w2r, b2r, w3r, b3r,
             gr, bbr, o_ref):
        nd = n_ref[...]
        agg = p0_ref[0] + p1_ref[0]
        h = _mm(jnp.concatenate([nd, agg], axis=1), w1r[...]) + b1r[...]
        h = jnp.maximum(h, 0.0)
        h = jnp.maximum(_mm(h, w2r[...]) + b2r[...], 0.0)
        h = _mm(h, w3r[...]) + b3r[...]
        o_ref[...] = nd + _ln(h, gr[...], bbr[...])

    z = lambda i: (0, 0)
    return pl.pallas_call(
        body,
        grid=(N // NB,),
        in_specs=[
            pl.BlockSpec((NB, 128), lambda i: (i, 0)),
            pl.BlockSpec((1, NB, 128), lambda i: (0, i, 0)),
            pl.BlockSpec((1, NB, 128), lambda i: (1, i, 0)),
            pl.BlockSpec((256, 128), z),
            pl.BlockSpec((1, 128), z),
            pl.BlockSpec((128, 128), z), pl.BlockSpec((1, 128), z),
            pl.BlockSpec((128, 128), z), pl.BlockSpec((1, 128), z),
            pl.BlockSpec((1, 128), z), pl.BlockSpec((1, 128), z),
        ],
        out_specs=pl.BlockSpec((NB, 128), lambda i: (i, 0)),
        out_shape=jax.ShapeDtypeStruct((N, 128), f32),
    )(nodes, p, p, w1, b1, w2, b2, w3, b3, g, bb)


def _decoder(nodes, w1, b1, w2, b2, w3p, b3p):
    def body(n_ref, w1r, b1r, w2r, b2r, w3r, b3r, o_ref):
        h = jnp.maximum(_mm(n_ref[...], w1r[...]) + b1r[...], 0.0)
        h = jnp.maximum(_mm(h, w2r[...]) + b2r[...], 0.0)
        o_ref[...] = _mm(h, w3r[...]) + b3r[...]

    z = lambda i: (0, 0)
    return pl.pallas_call(
        body,
        grid=(N // NB,),
        in_specs=[
            pl.BlockSpec((NB, 128), lambda i: (i, 0)),
            pl.BlockSpec((128, 128), z), pl.BlockSpec((1, 128), z),
            pl.BlockSpec((128, 128), z), pl.BlockSpec((1, 128), z),
            pl.BlockSpec((128, 128), z), pl.BlockSpec((1, 128), z),
        ],
        out_specs=pl.BlockSpec((NB, 128), lambda i: (i, 0)),
        out_shape=jax.ShapeDtypeStruct((N, 128), f32),
    )(nodes, w1, b1, w2, b2, w3p, b3p)


def _row(b):
    return b.reshape(1, -1)


def kernel(velocity, mesh_pos, node_type, cells, is_training, params):
    del is_training
    c = cells.astype(i32)
    ea = jnp.concatenate([c[:, 0], c[:, 1], c[:, 2]])
    eb = jnp.concatenate([c[:, 1], c[:, 2], c[:, 0]])
    lo = jnp.minimum(ea, eb)
    hi = jnp.maximum(ea, eb)
    pad = EH - E0
    lo_p = jnp.concatenate([lo, jnp.zeros((pad,), i32)])
    hi_p = jnp.concatenate([hi, jnp.zeros((pad,), i32)])
    key_p = jnp.concatenate([lo * N + hi, jnp.full((pad,), PADKEY, i32)])
    key3 = key_p.reshape(NT, NCH, CH)
    lo2 = lo_p.reshape(NT, PT)
    hi2 = hi_p.reshape(NT, PT)
    mx = mesh_pos[:, 0] + 0.0
    my = mesh_pos[:, 1] + 0.0
    zflat = jnp.zeros((PT * 8,), f32)

    tbl = _dedup_scatter(key3)
    featf, agga, aggb = _dedup_features(key3, lo2, hi2, mx, my, zflat, tbl)
    feat = featf.reshape(EH, 8)
    aggd = jnp.stack([agga.reshape(16, SNC, CH), aggb.reshape(16, SNC, CH)])
    lohi = jnp.stack([lo2.reshape(NT, NCH, CH), hi2.reshape(NT, NCH, CH)])
    zer = jnp.zeros((TR, 128), f32)

    nt1h = jax.nn.one_hot(node_type[:, 0], 9, dtype=f32)
    nf16 = jnp.concatenate([velocity, nt1h, jnp.zeros((N, 5), f32)], axis=1)

    def mlp3(p):
        (w1, b1), (w2, b2), (w3, b3) = p
        return w1, _row(b1), w2, _row(b2), w3, _row(b3)

    nw1, nb1, nw2, nb2, nw3, nb3 = mlp3(params['enc_node']['mlp'])
    nw1p = jnp.zeros((16, 128), f32).at[:11].set(nw1)
    ng, nbb = params['enc_node']['ln']
    nmean, nstd = _node_stats(nf16)
    nodes = _node_encoder(nf16, nmean, nstd, nw1p, nb1, nw2, nb2, nw3, nb3,
                          _row(ng), _row(nbb))

    ew1, eb1, ew2, eb2, ew3, eb3 = mlp3(params['enc_edge']['mlp'])
    ew1p = jnp.zeros((8, 128), f32).at[:3].set(ew1)
    eg, ebb = params['enc_edge']['ln']
    esums = _edge_sums(feat)
    edges = _edge_encoder(feat, esums, ew1p, eb1, ew2, eb2, ew3, eb3,
                          _row(eg), _row(ebb))

    for blk in params['blocks']:
        (w1, b1), (w2, b2), (w3, b3) = blk['edge_mlp']
        eg_, ebb_ = blk['edge_ln']
        (nw1_, nb1_), (nw2_, nb2_), (nw3_, nb3_) = blk['node_mlp']
        ng_, nbb_ = blk['node_ln']

        gl, gh = _gather_nodes(nodes, lohi)
        edges = _edge_mlp(edges, gl, gh, w1, _row(b1),
                          w2, _row(b2), w3, _row(b3), _row(eg_), _row(ebb_))
        p = _segment_sum(edges, aggd, zer)
        nodes = _node_mlp(nodes, p, nw1_, _row(nb1_),
                          nw2_, _row(nb2_), nw3_, _row(nb3_),
                          _row(ng_), _row(nbb_))

    (dw1, db1), (dw2, db2), (dw3, db3) = params['dec']['mlp']
    dw3p = jnp.zeros((128, 128), f32).at[:, :2].set(dw3)
    db3p = jnp.zeros((1, 128), f32).at[0, :2].set(db3)
    out = _decoder(nodes, dw1, _row(db1), dw2, _row(db2), dw3p, db3p)
    return out[:, :2]


# speedup vs baseline: 2.2157x; 1.0675x over previous
import functools

import jax
import jax.numpy as jnp
from jax import lax
from jax.experimental import pallas as pl
from jax.experimental.pallas import tpu as pltpu
from jax.experimental.pallas import tpu_sc as plsc

N = 10000
E0 = 60000
NT = 32
PT = 1920
EH = NT * PT
E = 2 * EH
CH = 128
NCH = PT // CH
TBL = 100_000_008
PADKEY = 100_000_000
DUMP = N
TR = 10240
SPT = 2 * PT
SNC = SPT // CH
RB = 512
NEB = E // RB
HEB = NEB // 2
NB = 1000

_mesh = plsc.VectorSubcoreMesh(core_axis_name="c", subcore_axis_name="s",
                               num_cores=2, num_subcores=16)
_sc_params = pltpu.CompilerParams(needs_layout_passes=False)
f32 = jnp.float32
i32 = jnp.int32


def _wid():
    return lax.axis_index("s") * 2 + lax.axis_index("c")


@functools.partial(
    pl.kernel,
    out_type=jax.ShapeDtypeStruct((TBL,), i32),
    mesh=_mesh,
    compiler_params=_sc_params,
    scratch_types=[
        pltpu.VMEM((NCH, CH), i32),
        pltpu.VMEM((PT,), i32),
        pltpu.SemaphoreType.DMA,
    ],
)
def _dedup_scatter(key_hbm, tbl_hbm, key_v, val_v, sem):
    wid = _wid()
    base = wid * PT
    pltpu.sync_copy(key_hbm.at[wid], key_v)

    def fill(t, c):
        val_v[pl.ds(t * 16, 16)] = lax.iota(i32, 16) + (base + t * 16)
        return c

    lax.fori_loop(0, PT // 16, fill, 0)

    def scat(j, c):
        pltpu.async_copy(
            val_v.at[pl.ds(j * CH, CH)], tbl_hbm.at[key_v.at[j]], sem
        ).wait()
        return c

    lax.fori_loop(0, NCH, scat, 0)


@functools.partial(
    pl.kernel,
    out_type=[
        jax.ShapeDtypeStruct((EH * 8,), f32),
        jax.ShapeDtypeStruct((EH,), i32),
        jax.ShapeDtypeStruct((EH,), i32),
    ],
    mesh=_mesh,
    compiler_params=_sc_params,
    scratch_types=[
        pltpu.VMEM((NCH, CH), i32),
        pltpu.VMEM((PT,), i32),
        pltpu.VMEM((PT,), i32),
        pltpu.VMEM((PT,), i32),
        pltpu.VMEM((N,), f32),
        pltpu.VMEM((N,), f32),
        pltpu.VMEM((PT * 8,), f32),
        pltpu.VMEM((PT,), i32),
        pltpu.VMEM((PT,), i32),
        pltpu.SemaphoreType.DMA,
    ],
)
def _dedup_features(key_hbm, lo_hbm, hi_hbm, mx_hbm, my_hbm, zf_hbm, tbl_hbm,
                    feat_hbm, agga_hbm, aggb_hbm,
                    key_v, lo_v, hi_v, w_v, mx_v, my_v, feat_v, aa_v, ab_v, sem):
    wid = _wid()
    base = wid * PT
    pltpu.sync_copy(key_hbm.at[wid], key_v)
    pltpu.sync_copy(lo_hbm.at[wid], lo_v)
    pltpu.sync_copy(hi_hbm.at[wid], hi_v)
    pltpu.sync_copy(mx_hbm, mx_v)
    pltpu.sync_copy(my_hbm, my_v)
    pltpu.sync_copy(zf_hbm, feat_v)

    def gat(j, c):
        pltpu.async_copy(
            tbl_hbm.at[key_v.at[j]], w_v.at[pl.ds(j * CH, CH)], sem
        ).wait()
        return c

    lax.fori_loop(0, NCH, gat, 0)

    def body(t, c):
        sl = pl.ds(t * 16, 16)
        lo16 = lo_v[sl]
        hi16 = hi_v[sl]
        w16 = w_v[sl]
        g16 = lax.iota(i32, 16) + (base + t * 16)
        rep = (w16 == g16) & (g16 < E0)
        ax = plsc.load_gather(mx_v, [lo16]) - plsc.load_gather(mx_v, [hi16])
        ay = plsc.load_gather(my_v, [lo16]) - plsc.load_gather(my_v, [hi16])
        n2 = ax * ax + ay * ay
        repf = jnp.where(rep, 1.0, 0.0).astype(f32)
        fb = (lax.iota(i32, 16) + t * 16) * 8
        plsc.store_scatter(feat_v, [fb], ax)
        plsc.store_scatter(feat_v, [fb + 1], ay)
        plsc.store_scatter(feat_v, [fb + 2], n2)
        plsc.store_scatter(feat_v, [fb + 3], repf)
        aa_v[sl] = jnp.where(rep, hi16, DUMP)
        ab_v[sl] = jnp.where(rep, lo16, DUMP)
        return c

    lax.fori_loop(0, PT // 16, body, 0)
    pltpu.sync_copy(feat_v, feat_hbm.at[pl.ds(base * 8, PT * 8)])
    pltpu.sync_copy(aa_v, agga_hbm.at[pl.ds(base, PT)])
    pltpu.sync_copy(ab_v, aggb_hbm.at[pl.ds(base, PT)])


@functools.partial(
    pl.kernel,
    out_type=jax.ShapeDtypeStruct((2, EH, 128), f32),
    mesh=_mesh,
    compiler_params=_sc_params,
    scratch_types=[
        pltpu.VMEM((2 * NCH, CH), i32),
        pltpu.VMEM((CH, 128), f32),
        pltpu.VMEM((CH, 128), f32),
        pltpu.SemaphoreType.DMA,
        pltpu.SemaphoreType.DMA,
        pltpu.SemaphoreType.DMA,
    ],
)
def _gather_nodes(nodes_hbm, lohi_hbm, g2_hbm, idx_v, ba, bb, sga, sgb, sw):
    wid = _wid()
    base = wid * PT
    pltpu.sync_copy(lohi_hbm.at[0, wid], idx_v.at[pl.ds(0, NCH)])
    pltpu.sync_copy(lohi_hbm.at[1, wid], idx_v.at[pl.ds(NCH, NCH)])

    def dst(j):
        return g2_hbm.at[j // NCH, pl.ds(base + (j % NCH) * CH, CH)]

    pltpu.async_copy(nodes_hbm.at[idx_v.at[0]], ba, sga)

    def body(jj, c):
        j0 = 2 * jj
        pltpu.async_copy(nodes_hbm.at[idx_v.at[j0 + 1]], bb, sgb)
        pltpu.make_async_copy(nodes_hbm.at[idx_v.at[j0]], ba, sga).wait()
        pltpu.sync_copy(ba, dst(j0))

        @pl.when(jj < NCH - 1)
        def _():
            pltpu.async_copy(nodes_hbm.at[idx_v.at[j0 + 2]], ba, sga)

        pltpu.make_async_copy(nodes_hbm.at[idx_v.at[j0 + 1]], bb, sgb).wait()
        pltpu.sync_copy(bb, dst(j0 + 1))
        return c

    lax.fori_loop(0, NCH, body, 0)


@functools.partial(
    pl.kernel,
    out_type=jax.ShapeDtypeStruct((2, TR, 128), f32),
    mesh=_mesh,
    compiler_params=_sc_params,
    scratch_types=[
        pltpu.VMEM((SNC, CH), i32),
        pltpu.VMEM((CH, 128), f32),
        pltpu.VMEM((CH, 128), f32),
        pltpu.VMEM_SHARED((TR, 128), f32),
        pltpu.SemaphoreType.DMA,
        pltpu.SemaphoreType.DMA,
    ],
)
def _segment_sum(edges_hbm, agg_hbm, zer_hbm, p_hbm,
                 idx_v, ebuf, ebuf2, tbl_s, sem, sem2):
    cid = lax.axis_index("c")
    sid = lax.axis_index("s")
    pltpu.sync_copy(agg_hbm.at[cid, sid], idx_v)
    pltpu.sync_copy(zer_hbm.at[pl.ds(sid * (TR // 16), TR // 16)],
                    tbl_s.at[pl.ds(sid * (TR // 16), TR // 16)])
    plsc.subcore_barrier()
    base = cid * EH + sid * SPT

    def src(j):
        return edges_hbm.at[pl.ds(base + j * CH, CH)]

    pltpu.async_copy(src(0), ebuf, sem)

    def body(jj, c):
        j0 = 2 * jj
        pltpu.async_copy(src(j0 + 1), ebuf2, sem2)
        pltpu.make_async_copy(src(j0), ebuf, sem).wait()
        pltpu.sync_copy(ebuf, tbl_s.at[idx_v.at[j0]], add=True)

        @pl.when(jj < SNC // 2 - 1)
        def _():
            pltpu.async_copy(src(j0 + 2), ebuf, sem)

        pltpu.make_async_copy(src(j0 + 1), ebuf2, sem2).wait()
        pltpu.sync_copy(ebuf2, tbl_s.at[idx_v.at[j0 + 1]], add=True)
        return c

    lax.fori_loop(0, SNC // 2, body, 0)
    plsc.subcore_barrier()
    rpt = TR // 16
    pltpu.sync_copy(tbl_s.at[pl.ds(sid * rpt, rpt)],
                    p_hbm.at[cid, pl.ds(sid * rpt, rpt)])


def _ln(h, g, bb):
    mu = jnp.mean(h, axis=-1, keepdims=True)
    d = h - mu
    var = jnp.mean(d * d, axis=-1, keepdims=True)
    return d / jnp.sqrt(var + 1e-5) * g + bb


def _node_stats(nf16):
    def body(x_ref, mean_ref, std_ref):
        x = x_ref[...]
        mu = jnp.mean(x, axis=0, keepdims=True)
        ex2 = jnp.mean(x * x, axis=0, keepdims=True)
        sd = jnp.sqrt(jnp.maximum(ex2 - mu * mu, 0.0))
        mean_ref[...] = mu
        std_ref[...] = jnp.maximum(sd, 1e-8)

    return pl.pallas_call(
        body,
        out_shape=[jax.ShapeDtypeStruct((1, 16), f32),
                   jax.ShapeDtypeStruct((1, 16), f32)],
    )(nf16)


def _edge_sums(feat):
    def body(f_ref, o_ref):
        i = pl.program_id(0)
        x = f_ref[...]
        rx = x[:, 0:1]
        ry = x[:, 1:2]
        n2 = x[:, 2:3]
        nm = jnp.sqrt(n2)
        rp = x[:, 3:4]
        s0 = jnp.sum(rp)
        s1 = jnp.sum(nm * rp)
        s2 = jnp.sum(rx * rx * rp)
        s3 = jnp.sum(ry * ry * rp)
        s4 = jnp.sum(n2 * rp)
        col = lax.broadcasted_iota(i32, (1, 8), 1)
        vals = jnp.where(
            col == 0, s0,
            jnp.where(col == 1, s1,
                      jnp.where(col == 2, s2,
                                jnp.where(col == 3, s3,
                                          jnp.where(col == 4, s4, 0.0)))))
        o_ref[...] = jnp.where(i == 0, vals, o_ref[...] + vals)

    return pl.pallas_call(
        body,
        grid=(EH // RB,),
        in_specs=[pl.BlockSpec((RB, 8), lambda i: (i, 0))],
        out_specs=pl.BlockSpec((1, 8), lambda i: (0, 0)),
        out_shape=jax.ShapeDtypeStruct((1, 8), f32),
    )(feat)


def _mm(a, b):
    return jnp.dot(a, b, preferred_element_type=f32)


def _node_encoder(nf16, mean, std, w1, b1, w2, b2, w3, b3, g, bb):
    def body(x_ref, m_ref, s_ref, w1r, b1r, w2r, b2r, w3r, b3r, gr, bbr, o_ref):
        x = (x_ref[...] - m_ref[...]) / s_ref[...]
        h = jnp.maximum(_mm(x, w1r[...]) + b1r[...], 0.0)
        h = jnp.maximum(_mm(h, w2r[...]) + b2r[...], 0.0)
        h = _mm(h, w3r[...]) + b3r[...]
        o_ref[...] = _ln(h, gr[...], bbr[...])

    z = lambda i: (0, 0)
    return pl.pallas_call(
        body,
        grid=(N // NB,),
        in_specs=[
            pl.BlockSpec((NB, 16), lambda i: (i, 0)),
            pl.BlockSpec((1, 16), z), pl.BlockSpec((1, 16), z),
            pl.BlockSpec((16, 128), z), pl.BlockSpec((1, 128), z),
            pl.BlockSpec((128, 128), z), pl.BlockSpec((1, 128), z),
            pl.BlockSpec((128, 128), z), pl.BlockSpec((1, 128), z),
            pl.BlockSpec((1, 128), z), pl.BlockSpec((1, 128), z),
        ],
        out_specs=pl.BlockSpec((NB, 128), lambda i: (i, 0)),
        out_shape=jax.ShapeDtypeStruct((N, 128), f32),
    )(nf16, mean, std, w1, b1, w2, b2, w3, b3, g, bb)


def _edge_encoder(feat, sums, w1, b1, w2, b2, w3, b3, g, bb):
    def body(f_ref, s_ref, w1r, b1r, w2r, b2r, w3r, b3r, gr, bbr, o_ref):
        i = pl.program_id(0)
        cnt = s_ref[0, 0]
        sn = s_ref[0, 1] / cnt
        sdx = jnp.maximum(jnp.sqrt(jnp.maximum(s_ref[0, 2] / cnt, 0.0)), 1e-8)
        sdy = jnp.maximum(jnp.sqrt(jnp.maximum(s_ref[0, 3] / cnt, 0.0)), 1e-8)
        sdn = jnp.maximum(
            jnp.sqrt(jnp.maximum(s_ref[0, 4] / cnt - sn * sn, 0.0)), 1e-8)
        col = lax.broadcasted_iota(i32, (1, 8), 1)
        mean = jnp.where(col == 2, sn, 0.0)
        std = jnp.where(
            col == 0, sdx, jnp.where(col == 1, sdy, jnp.where(col == 2, sdn, 1.0)))
        sgn = jnp.where(i < HEB, 1.0, -1.0)
        sv = jnp.where(col < 2, sgn, 1.0)
        f = f_ref[...]
        f = jnp.where(col == 2, jnp.sqrt(jnp.maximum(f, 0.0)), f)
        x = (f - mean) / std * sv
        h = jnp.maximum(_mm(x, w1r[...]) + b1r[...], 0.0)
        h = jnp.maximum(_mm(h, w2r[...]) + b2r[...], 0.0)
        h = _mm(h, w3r[...]) + b3r[...]
        o_ref[...] = _ln(h, gr[...], bbr[...])

    z = lambda i: (0, 0)
    return pl.pallas_call(
        body,
        grid=(NEB,),
        in_specs=[
            pl.BlockSpec((RB, 8), lambda i: (i % HEB, 0)),
            pl.BlockSpec((1, 8), z),
            pl.BlockSpec((8, 128), z), pl.BlockSpec((1, 128), z),
            pl.BlockSpec((128, 128), z), pl.BlockSpec((1, 128), z),
            pl.BlockSpec((128, 128), z), pl.BlockSpec((1, 128), z),
            pl.BlockSpec((1, 128), z), pl.BlockSpec((1, 128), z),
        ],
        out_specs=pl.BlockSpec((RB, 128), lambda i: (i, 0)),
        out_shape=jax.ShapeDtypeStruct((E, 128), f32),
    )(feat, sums, w1, b1, w2, b2, w3, b3, g, bb)


def _edge_mlp(edges, g2, w1, b1, w2, b2, w3, b3, g, bb):
    def body(e_ref, l_ref, h_ref, w1r, b1r, w2r, b2r, w3r, b3r,
             gr, bbr, o_ref):
        i = pl.program_id(0)
        first = i < HEB
        lv = l_ref[0]
        hv = h_ref[0]
        s = jnp.where(first, lv, hv)
        r = jnp.where(first, hv, lv)
        e = e_ref[...]
        h = _mm(jnp.concatenate([e, s, r], axis=1), w1r[...]) + b1r[...]
        h = jnp.maximum(h, 0.0)
        h = jnp.maximum(_mm(h, w2r[...]) + b2r[...], 0.0)
        h = _mm(h, w3r[...]) + b3r[...]
        o_ref[...] = e + _ln(h, gr[...], bbr[...])

    z = lambda i: (0, 0)
    return pl.pallas_call(
        body,
        grid=(NEB,),
        in_specs=[
            pl.BlockSpec((RB, 128), lambda i: (i, 0)),
            pl.BlockSpec((1, RB, 128), lambda i: (0, i % HEB, 0)),
            pl.BlockSpec((1, RB, 128), lambda i: (1, i % HEB, 0)),
            pl.BlockSpec((384, 128), z), pl.BlockSpec((1, 128), z),
            pl.BlockSpec((128, 128), z), pl.BlockSpec((1, 128), z),
            pl.BlockSpec((128, 128), z), pl.BlockSpec((1, 128), z),
            pl.BlockSpec((1, 128), z), pl.BlockSpec((1, 128), z),
        ],
        out_specs=pl.BlockSpec((RB, 128), lambda i: (i, 0)),
        out_shape=jax.ShapeDtypeStruct((E, 128), f32),
    )(edges, g2, g2, w1, b1, w2, b2, w3, b3, g, bb)


def _node_mlp(nodes, p, w1, b1, w2, b2, w3, b3, g, bb):
    def body(n_ref, p0_ref, p1_ref, w1r, b1r, w2r, b2r, w3r, b3r,
             gr, bbr, o_ref):
        nd = n_ref[...]
        agg = p0_ref[0] + p1_ref[0]
        h = _mm(jnp.concatenate([nd, agg], axis=1), w1r[...]) + b1r[...]
        h = jnp.maximum(h, 0.0)
        h = jnp.maximum(_mm(h, w2r[...]) + b2r[...], 0.0)
        h = _mm(h, w3r[...]) + b3r[...]
        o_ref[...] = nd + _ln(h, gr[...], bbr[...])

    z = lambda i: (0, 0)
    return pl.pallas_call(
        body,
        grid=(N // NB,),
        in_specs=[
            pl.BlockSpec((NB, 128), lambda i: (i, 0)),
            pl.BlockSpec((1, NB, 128), lambda i: (0, i, 0)),
            pl.BlockSpec((1, NB, 128), lambda i: (1, i, 0)),
            pl.BlockSpec((256, 128), z),
            pl.BlockSpec((1, 128), z),
            pl.BlockSpec((128, 128), z), pl.BlockSpec((1, 128), z),
            pl.BlockSpec((128, 128), z), pl.BlockSpec((1, 128), z),
            pl.BlockSpec((1, 128), z), pl.BlockSpec((1, 128), z),
        ],
        out_specs=pl.BlockSpec((NB, 128), lambda i: (i, 0)),
        out_shape=jax.ShapeDtypeStruct((N, 128), f32),
    )(nodes, p, p, w1, b1, w2, b2, w3, b3, g, bb)


def _decoder(nodes, w1, b1, w2, b2, w3p, b3p):
    def body(n_ref, w1r, b1r, w2r, b2r, w3r, b3r, o_ref):
        h = jnp.maximum(_mm(n_ref[...], w1r[...]) + b1r[...], 0.0)
        h = jnp.maximum(_mm(h, w2r[...]) + b2r[...], 0.0)
        o_ref[...] = _mm(h, w3r[...]) + b3r[...]

    z = lambda i: (0, 0)
    return pl.pallas_call(
        body,
        grid=(N // NB,),
        in_specs=[
            pl.BlockSpec((NB, 128), lambda i: (i, 0)),
            pl.BlockSpec((128, 128), z), pl.BlockSpec((1, 128), z),
            pl.BlockSpec((128, 128), z), pl.BlockSpec((1, 128), z),
            pl.BlockSpec((128, 128), z), pl.BlockSpec((1, 128), z),
        ],
        out_specs=pl.BlockSpec((NB, 128), lambda i: (i, 0)),
        out_shape=jax.ShapeDtypeStruct((N, 128), f32),
    )(nodes, w1, b1, w2, b2, w3p, b3p)


def _row(b):
    return b.reshape(1, -1)


def kernel(velocity, mesh_pos, node_type, cells, is_training, params):
    del is_training
    c = cells.astype(i32)
    ea = jnp.concatenate([c[:, 0], c[:, 1], c[:, 2]])
    eb = jnp.concatenate([c[:, 1], c[:, 2], c[:, 0]])
    lo = jnp.minimum(ea, eb)
    hi = jnp.maximum(ea, eb)
    pad = EH - E0
    lo_p = jnp.concatenate([lo, jnp.zeros((pad,), i32)])
    hi_p = jnp.concatenate([hi, jnp.zeros((pad,), i32)])
    key_p = jnp.concatenate([lo * N + hi, jnp.full((pad,), PADKEY, i32)])
    key3 = key_p.reshape(NT, NCH, CH)
    lo2 = lo_p.reshape(NT, PT)
    hi2 = hi_p.reshape(NT, PT)
    mx = mesh_pos[:, 0] + 0.0
    my = mesh_pos[:, 1] + 0.0
    zflat = jnp.zeros((PT * 8,), f32)

    tbl = _dedup_scatter(key3)
    featf, agga, aggb = _dedup_features(key3, lo2, hi2, mx, my, zflat, tbl)
    feat = featf.reshape(EH, 8)
    aggd = jnp.stack([agga.reshape(16, SNC, CH), aggb.reshape(16, SNC, CH)])
    lohi = jnp.stack([lo2.reshape(NT, NCH, CH), hi2.reshape(NT, NCH, CH)])
    zer = jnp.zeros((TR, 128), f32)

    nt1h = jax.nn.one_hot(node_type[:, 0], 9, dtype=f32)
    nf16 = jnp.concatenate([velocity, nt1h, jnp.zeros((N, 5), f32)], axis=1)

    def mlp3(p):
        (w1, b1), (w2, b2), (w3, b3) = p
        return w1, _row(b1), w2, _row(b2), w3, _row(b3)

    nw1, nb1, nw2, nb2, nw3, nb3 = mlp3(params['enc_node']['mlp'])
    nw1p = jnp.zeros((16, 128), f32).at[:11].set(nw1)
    ng, nbb = params['enc_node']['ln']
    nmean, nstd = _node_stats(nf16)
    nodes = _node_encoder(nf16, nmean, nstd, nw1p, nb1, nw2, nb2, nw3, nb3,
                          _row(ng), _row(nbb))

    ew1, eb1, ew2, eb2, ew3, eb3 = mlp3(params['enc_edge']['mlp'])
    ew1p = jnp.zeros((8, 128), f32).at[:3].set(ew1)
    eg, ebb = params['enc_edge']['ln']
    esums = _edge_sums(feat)
    edges = _edge_encoder(feat, esums, ew1p, eb1, ew2, eb2, ew3, eb3,
                          _row(eg), _row(ebb))

    for blk in params['blocks']:
        (w1, b1), (w2, b2), (w3, b3) = blk['edge_mlp']
        eg_, ebb_ = blk['edge_ln']
        (nw1_, nb1_), (nw2_, nb2_), (nw3_, nb3_) = blk['node_mlp']
        ng_, nbb_ = blk['node_ln']

        g2 = _gather_nodes(nodes, lohi)
        edges = _edge_mlp(edges, g2, w1, _row(b1),
                          w2, _row(b2), w3, _row(b3), _row(eg_), _row(ebb_))
        p = _segment_sum(edges, aggd, zer)
        nodes = _node_mlp(nodes, p, nw1_, _row(nb1_),
                          nw2_, _row(nb2_), nw3_, _row(nb3_),
                          _row(ng_), _row(nbb_))

    (dw1, db1), (dw2, db2), (dw3, db3) = params['dec']['mlp']
    dw3p = jnp.zeros((128, 128), f32).at[:, :2].set(dw3)
    db3p = jnp.zeros((1, 128), f32).at[0, :2].set(db3)
    out = _decoder(nodes, dw1, _row(db1), dw2, _row(db2), dw3p, db3p)
    return out[:, :2]


# speedup vs baseline: 2.6900x; 1.2141x over previous
import functools

import jax
import jax.numpy as jnp
from jax import lax
from jax.experimental import pallas as pl
from jax.experimental.pallas import tpu as pltpu
from jax.experimental.pallas import tpu_sc as plsc

N = 10000
E0 = 60000
NT = 32
PT = 1920
EH = NT * PT
E = 2 * EH
CH = 128
NCH = PT // CH
TBL = 100_000_008
PADKEY = 100_000_000
DUMP = N
TR = 10240
SPT = 2 * PT
SNC = SPT // CH
RB = 512
HEB = EH // RB
GC = 128
NGC = 2 * PT // GC
NSC = SPT // GC
NB = 1000

_mesh = plsc.VectorSubcoreMesh(core_axis_name="c", subcore_axis_name="s",
                               num_cores=2, num_subcores=16)
_sc_params = pltpu.CompilerParams(needs_layout_passes=False)
f32 = jnp.float32
i32 = jnp.int32


def _wid():
    return lax.axis_index("s") * 2 + lax.axis_index("c")


@functools.partial(
    pl.kernel,
    out_type=jax.ShapeDtypeStruct((TBL,), i32),
    mesh=_mesh,
    compiler_params=_sc_params,
    scratch_types=[
        pltpu.VMEM((NCH, CH), i32),
        pltpu.VMEM((PT,), i32),
        pltpu.SemaphoreType.DMA,
    ],
)
def _dedup_scatter(key_hbm, tbl_hbm, key_v, val_v, sem):
    wid = _wid()
    base = wid * PT
    pltpu.sync_copy(key_hbm.at[wid], key_v)

    def fill(t, c):
        val_v[pl.ds(t * 16, 16)] = lax.iota(i32, 16) + (base + t * 16)
        return c

    lax.fori_loop(0, PT // 16, fill, 0)

    def scat(j, c):
        pltpu.async_copy(
            val_v.at[pl.ds(j * CH, CH)], tbl_hbm.at[key_v.at[j]], sem
        ).wait()
        return c

    lax.fori_loop(0, NCH, scat, 0)


@functools.partial(
    pl.kernel,
    out_type=[
        jax.ShapeDtypeStruct((EH * 8,), f32),
        jax.ShapeDtypeStruct((EH,), i32),
        jax.ShapeDtypeStruct((EH,), i32),
    ],
    mesh=_mesh,
    compiler_params=_sc_params,
    scratch_types=[
        pltpu.VMEM((NCH, CH), i32),
        pltpu.VMEM((PT,), i32),
        pltpu.VMEM((PT,), i32),
        pltpu.VMEM((PT,), i32),
        pltpu.VMEM((N,), f32),
        pltpu.VMEM((N,), f32),
        pltpu.VMEM((PT * 8,), f32),
        pltpu.VMEM((PT,), i32),
        pltpu.VMEM((PT,), i32),
        pltpu.SemaphoreType.DMA,
    ],
)
def _dedup_features(key_hbm, lo_hbm, hi_hbm, mx_hbm, my_hbm, zf_hbm, tbl_hbm,
                    feat_hbm, agga_hbm, aggb_hbm,
                    key_v, lo_v, hi_v, w_v, mx_v, my_v, feat_v, aa_v, ab_v, sem):
    wid = _wid()
    base = wid * PT
    pltpu.sync_copy(key_hbm.at[wid], key_v)
    pltpu.sync_copy(lo_hbm.at[wid], lo_v)
    pltpu.sync_copy(hi_hbm.at[wid], hi_v)
    pltpu.sync_copy(mx_hbm, mx_v)
    pltpu.sync_copy(my_hbm, my_v)
    pltpu.sync_copy(zf_hbm, feat_v)

    def gat(j, c):
        pltpu.async_copy(
            tbl_hbm.at[key_v.at[j]], w_v.at[pl.ds(j * CH, CH)], sem
        ).wait()
        return c

    lax.fori_loop(0, NCH, gat, 0)

    def body(t, c):
        sl = pl.ds(t * 16, 16)
        lo16 = lo_v[sl]
        hi16 = hi_v[sl]
        w16 = w_v[sl]
        g16 = lax.iota(i32, 16) + (base + t * 16)
        rep = (w16 == g16) & (g16 < E0)
        ax = plsc.load_gather(mx_v, [lo16]) - plsc.load_gather(mx_v, [hi16])
        ay = plsc.load_gather(my_v, [lo16]) - plsc.load_gather(my_v, [hi16])
        n2 = ax * ax + ay * ay
        repf = jnp.where(rep, 1.0, 0.0).astype(f32)
        fb = (lax.iota(i32, 16) + t * 16) * 8
        plsc.store_scatter(feat_v, [fb], ax)
        plsc.store_scatter(feat_v, [fb + 1], ay)
        plsc.store_scatter(feat_v, [fb + 2], n2)
        plsc.store_scatter(feat_v, [fb + 3], repf)
        aa_v[sl] = jnp.where(rep, hi16, DUMP)
        ab_v[sl] = jnp.where(rep, lo16, DUMP)
        return c

    lax.fori_loop(0, PT // 16, body, 0)
    pltpu.sync_copy(feat_v, feat_hbm.at[pl.ds(base * 8, PT * 8)])
    pltpu.sync_copy(aa_v, agga_hbm.at[pl.ds(base, PT)])
    pltpu.sync_copy(ab_v, aggb_hbm.at[pl.ds(base, PT)])


@functools.partial(
    pl.kernel,
    out_type=jax.ShapeDtypeStruct((2, EH, 128), f32),
    mesh=_mesh,
    compiler_params=_sc_params,
    scratch_types=[
        pltpu.VMEM((NGC, GC), i32),
        pltpu.VMEM((GC, 128), f32),
        pltpu.VMEM((GC, 128), f32),
        pltpu.VMEM((GC, 128), f32),
        pltpu.VMEM((GC, 128), f32),
        pltpu.SemaphoreType.DMA,
        pltpu.SemaphoreType.DMA,
        pltpu.SemaphoreType.DMA,
        pltpu.SemaphoreType.DMA,
    ],
)
def _gather_nodes(nodes_hbm, lohi_hbm, g2_hbm, idx_v,
                  b0, b1, b2, b3, s0, s1, s2, s3):
    wid = _wid()
    base = wid * PT
    hpt = NGC // 2
    pltpu.sync_copy(lohi_hbm.at[0, wid], idx_v.at[pl.ds(0, hpt)])
    pltpu.sync_copy(lohi_hbm.at[1, wid], idx_v.at[pl.ds(hpt, hpt)])
    bufs = (b0, b1, b2, b3)
    sems = (s0, s1, s2, s3)

    def dst(j):
        return g2_hbm.at[j // hpt, pl.ds(base + (j % hpt) * GC, GC)]

    for j in range(3):
        pltpu.async_copy(nodes_hbm.at[idx_v.at[j]], bufs[j], sems[j])
    for j in range(NGC):
        k = j % 4
        pltpu.make_async_copy(nodes_hbm.at[idx_v.at[j]], bufs[k], sems[k]).wait()
        pltpu.sync_copy(bufs[k], dst(j))
        if j + 3 < NGC:
            kn = (j + 3) % 4
            pltpu.async_copy(nodes_hbm.at[idx_v.at[j + 3]], bufs[kn], sems[kn])


@functools.partial(
    pl.kernel,
    out_type=jax.ShapeDtypeStruct((2, TR, 128), f32),
    mesh=_mesh,
    compiler_params=_sc_params,
    scratch_types=[
        pltpu.VMEM((NSC, GC), i32),
        pltpu.VMEM((GC, 128), f32),
        pltpu.VMEM((GC, 128), f32),
        pltpu.VMEM_SHARED((TR, 128), f32),
        pltpu.SemaphoreType.DMA,
        pltpu.SemaphoreType.DMA,
    ],
)
def _segment_sum(edges_hbm, agg_hbm, zer_hbm, p_hbm,
                 idx_v, ebuf, ebuf2, tbl_s, sem, sem2):
    cid = lax.axis_index("c")
    sid = lax.axis_index("s")
    pltpu.sync_copy(agg_hbm.at[cid, sid], idx_v)
    pltpu.sync_copy(zer_hbm.at[pl.ds(sid * (TR // 16), TR // 16)],
                    tbl_s.at[pl.ds(sid * (TR // 16), TR // 16)])
    plsc.subcore_barrier()
    base = sid * SPT

    def src(j):
        return edges_hbm.at[cid, pl.ds(base + j * GC, GC)]

    ebufs = (ebuf, ebuf2)
    esems = (sem, sem2)
    pltpu.async_copy(src(0), ebufs[0], esems[0])
    for j in range(NSC):
        k = j % 2
        pltpu.make_async_copy(src(j), ebufs[k], esems[k]).wait()
        if j + 1 < NSC:
            kn = (j + 1) % 2
            pltpu.async_copy(src(j + 1), ebufs[kn], esems[kn])
        pltpu.sync_copy(ebufs[k], tbl_s.at[idx_v.at[j]], add=True)
    plsc.subcore_barrier()
    rpt = TR // 16
    pltpu.sync_copy(tbl_s.at[pl.ds(sid * rpt, rpt)],
                    p_hbm.at[cid, pl.ds(sid * rpt, rpt)])


def _ln(h, g, bb):
    mu = jnp.mean(h, axis=-1, keepdims=True)
    d = h - mu
    var = jnp.mean(d * d, axis=-1, keepdims=True)
    return d / jnp.sqrt(var + 1e-5) * g + bb


def _node_stats(nf16):
    def body(x_ref, mean_ref, std_ref):
        x = x_ref[...]
        mu = jnp.mean(x, axis=0, keepdims=True)
        ex2 = jnp.mean(x * x, axis=0, keepdims=True)
        sd = jnp.sqrt(jnp.maximum(ex2 - mu * mu, 0.0))
        mean_ref[...] = mu
        std_ref[...] = jnp.maximum(sd, 1e-8)

    return pl.pallas_call(
        body,
        out_shape=[jax.ShapeDtypeStruct((1, 16), f32),
                   jax.ShapeDtypeStruct((1, 16), f32)],
    )(nf16)


def _edge_sums(feat):
    def body(f_ref, o_ref):
        i = pl.program_id(0)
        x = f_ref[...]
        rx = x[:, 0:1]
        ry = x[:, 1:2]
        n2 = x[:, 2:3]
        nm = jnp.sqrt(n2)
        rp = x[:, 3:4]
        s0 = jnp.sum(rp)
        s1 = jnp.sum(nm * rp)
        s2 = jnp.sum(rx * rx * rp)
        s3 = jnp.sum(ry * ry * rp)
        s4 = jnp.sum(n2 * rp)
        col = lax.broadcasted_iota(i32, (1, 8), 1)
        vals = jnp.where(
            col == 0, s0,
            jnp.where(col == 1, s1,
                      jnp.where(col == 2, s2,
                                jnp.where(col == 3, s3,
                                          jnp.where(col == 4, s4, 0.0)))))
        o_ref[...] = jnp.where(i == 0, vals, o_ref[...] + vals)

    return pl.pallas_call(
        body,
        grid=(EH // RB,),
        in_specs=[pl.BlockSpec((RB, 8), lambda i: (i, 0))],
        out_specs=pl.BlockSpec((1, 8), lambda i: (0, 0)),
        out_shape=jax.ShapeDtypeStruct((1, 8), f32),
    )(feat)


def _mm(a, b):
    return jnp.dot(a, b, preferred_element_type=f32)


def _node_encoder(nf16, mean, std, w1, b1, w2, b2, w3, b3, g, bb):
    def body(x_ref, m_ref, s_ref, w1r, b1r, w2r, b2r, w3r, b3r, gr, bbr, o_ref):
        x = (x_ref[...] - m_ref[...]) / s_ref[...]
        h = jnp.maximum(_mm(x, w1r[...]) + b1r[...], 0.0)
        h = jnp.maximum(_mm(h, w2r[...]) + b2r[...], 0.0)
        h = _mm(h, w3r[...]) + b3r[...]
        o_ref[...] = _ln(h, gr[...], bbr[...])

    z = lambda i: (0, 0)
    return pl.pallas_call(
        body,
        grid=(N // NB,),
        in_specs=[
            pl.BlockSpec((NB, 16), lambda i: (i, 0)),
            pl.BlockSpec((1, 16), z), pl.BlockSpec((1, 16), z),
            pl.BlockSpec((16, 128), z), pl.BlockSpec((1, 128), z),
            pl.BlockSpec((128, 128), z), pl.BlockSpec((1, 128), z),
            pl.BlockSpec((128, 128), z), pl.BlockSpec((1, 128), z),
            pl.BlockSpec((1, 128), z), pl.BlockSpec((1, 128), z),
        ],
        out_specs=pl.BlockSpec((NB, 128), lambda i: (i, 0)),
        out_shape=jax.ShapeDtypeStruct((N, 128), f32),
    )(nf16, mean, std, w1, b1, w2, b2, w3, b3, g, bb)


def _edge_encoder(feat, sums, w1, b1, w2, b2, w3, b3, g, bb):
    def body(f_ref, s_ref, w1r, b1r, w2r, b2r, w3r, b3r, gr, bbr, o_ref):
        cnt = s_ref[0, 0]
        sn = s_ref[0, 1] / cnt
        sdx = jnp.maximum(jnp.sqrt(jnp.maximum(s_ref[0, 2] / cnt, 0.0)), 1e-8)
        sdy = jnp.maximum(jnp.sqrt(jnp.maximum(s_ref[0, 3] / cnt, 0.0)), 1e-8)
        sdn = jnp.maximum(
            jnp.sqrt(jnp.maximum(s_ref[0, 4] / cnt - sn * sn, 0.0)), 1e-8)
        col = lax.broadcasted_iota(i32, (1, 8), 1)
        mean = jnp.where(col == 2, sn, 0.0)
        std = jnp.where(
            col == 0, sdx, jnp.where(col == 1, sdy, jnp.where(col == 2, sdn, 1.0)))
        sv = jnp.where(col < 2, -1.0, 1.0)
        f = f_ref[...]
        f = jnp.where(col == 2, jnp.sqrt(jnp.maximum(f, 0.0)), f)
        x = (f - mean) / std

        def enc(xx):
            h = jnp.maximum(_mm(xx, w1r[...]) + b1r[...], 0.0)
            h = jnp.maximum(_mm(h, w2r[...]) + b2r[...], 0.0)
            h = _mm(h, w3r[...]) + b3r[...]
            return _ln(h, gr[...], bbr[...])

        o_ref[0] = enc(x)
        o_ref[1] = enc(x * sv)

    z = lambda i: (0, 0)
    return pl.pallas_call(
        body,
        grid=(HEB,),
        in_specs=[
            pl.BlockSpec((RB, 8), lambda i: (i, 0)),
            pl.BlockSpec((1, 8), z),
            pl.BlockSpec((8, 128), z), pl.BlockSpec((1, 128), z),
            pl.BlockSpec((128, 128), z), pl.BlockSpec((1, 128), z),
            pl.BlockSpec((128, 128), z), pl.BlockSpec((1, 128), z),
            pl.BlockSpec((1, 128), z), pl.BlockSpec((1, 128), z),
        ],
        out_specs=pl.BlockSpec((2, RB, 128), lambda i: (0, i, 0)),
        out_shape=jax.ShapeDtypeStruct((2, EH, 128), f32),
    )(feat, sums, w1, b1, w2, b2, w3, b3, g, bb)


def _edge_mlp(edges, g2, w1, b1, w2, b2, w3, b3, g, bb):
    def body(e_ref, l_ref, h_ref, w1r, b1r, w2r, b2r, w3r, b3r,
             gr, bbr, o_ref):
        lv = l_ref[0]
        hv = h_ref[0]

        def mlp(e, sr, rr):
            h = _mm(jnp.concatenate([e, sr, rr], axis=1), w1r[...]) + b1r[...]
            h = jnp.maximum(h, 0.0)
            h = jnp.maximum(_mm(h, w2r[...]) + b2r[...], 0.0)
            h = _mm(h, w3r[...]) + b3r[...]
            return e + _ln(h, gr[...], bbr[...])

        o_ref[0] = mlp(e_ref[0], lv, hv)
        o_ref[1] = mlp(e_ref[1], hv, lv)

    z = lambda i: (0, 0)
    return pl.pallas_call(
        body,
        grid=(HEB,),
        in_specs=[
            pl.BlockSpec((2, RB, 128), lambda i: (0, i, 0)),
            pl.BlockSpec((1, RB, 128), lambda i: (0, i, 0)),
            pl.BlockSpec((1, RB, 128), lambda i: (1, i, 0)),
            pl.BlockSpec((384, 128), z), pl.BlockSpec((1, 128), z),
            pl.BlockSpec((128, 128), z), pl.BlockSpec((1, 128), z),
            pl.BlockSpec((128, 128), z), pl.BlockSpec((1, 128), z),
            pl.BlockSpec((1, 128), z), pl.BlockSpec((1, 128), z),
        ],
        out_specs=pl.BlockSpec((2, RB, 128), lambda i: (0, i, 0)),
        out_shape=jax.ShapeDtypeStruct((2, EH, 128), f32),
    )(edges, g2, g2, w1, b1, w2, b2, w3, b3, g, bb)


def _node_mlp(nodes, p, w1, b1, w2, b2, w3, b3, g, bb):
    def body(n_ref, p0_ref, p1_ref, w1r, b1r, w2r, b2r, w3r, b3r,
             gr, bbr, o_ref):
        nd = n_ref[...]
        agg = p0_ref[0] + p1_ref[0]
        h = _mm(jnp.concatenate([nd, agg], axis=1), w1r[...]) + b1r[...]
        h = jnp.maximum(h, 0.0)
        h = jnp.maximum(_mm(h, w2r[...]) + b2r[...], 0.0)
        h = _mm(h, w3r[...]) + b3r[...]
        o_ref[...] = nd + _ln(h, gr[...], bbr[...])

    z = lambda i: (0, 0)
    return pl.pallas_call(
        body,
        grid=(N // NB,),
        in_specs=[
            pl.BlockSpec((NB, 128), lambda i: (i, 0)),
            pl.BlockSpec((1, NB, 128), lambda i: (0, i, 0)),
            pl.BlockSpec((1, NB, 128), lambda i: (1, i, 0)),
            pl.BlockSpec((256, 128), z),
            pl.BlockSpec((1, 128), z),
            pl.BlockSpec((128, 128), z), pl.BlockSpec((1, 128), z),
            pl.BlockSpec((128, 128), z), pl.BlockSpec((1, 128), z),
            pl.BlockSpec((1, 128), z), pl.BlockSpec((1, 128), z),
        ],
        out_specs=pl.BlockSpec((NB, 128), lambda i: (i, 0)),
        out_shape=jax.ShapeDtypeStruct((N, 128), f32),
    )(nodes, p, p, w1, b1, w2, b2, w3, b3, g, bb)


def _decoder(nodes, w1, b1, w2, b2, w3p, b3p):
    def body(n_ref, w1r, b1r, w2r, b2r, w3r, b3r, o_ref):
        h = jnp.maximum(_mm(n_ref[...], w1r[...]) + b1r[...], 0.0)
        h = jnp.maximum(_mm(h, w2r[...]) + b2r[...], 0.0)
        o_ref[...] = _mm(h, w3r[...]) + b3r[...]

    z = lambda i: (0, 0)
    return pl.pallas_call(
        body,
        grid=(N // NB,),
        in_specs=[
            pl.BlockSpec((NB, 128), lambda i: (i, 0)),
            pl.BlockSpec((128, 128), z), pl.BlockSpec((1, 128), z),
            pl.BlockSpec((128, 128), z), pl.BlockSpec((1, 128), z),
            pl.BlockSpec((128, 128), z), pl.BlockSpec((1, 128), z),
        ],
        out_specs=pl.BlockSpec((NB, 128), lambda i: (i, 0)),
        out_shape=jax.ShapeDtypeStruct((N, 128), f32),
    )(nodes, w1, b1, w2, b2, w3p, b3p)


def _row(b):
    return b.reshape(1, -1)


def kernel(velocity, mesh_pos, node_type, cells, is_training, params):
    del is_training
    c = cells.astype(i32)
    ea = jnp.concatenate([c[:, 0], c[:, 1], c[:, 2]])
    eb = jnp.concatenate([c[:, 1], c[:, 2], c[:, 0]])
    lo = jnp.minimum(ea, eb)
    hi = jnp.maximum(ea, eb)
    pad = EH - E0
    lo_p = jnp.concatenate([lo, jnp.zeros((pad,), i32)])
    hi_p = jnp.concatenate([hi, jnp.zeros((pad,), i32)])
    key_p = jnp.concatenate([lo * N + hi, jnp.full((pad,), PADKEY, i32)])
    key3 = key_p.reshape(NT, NCH, CH)
    lo2 = lo_p.reshape(NT, PT)
    hi2 = hi_p.reshape(NT, PT)
    mx = mesh_pos[:, 0] + 0.0
    my = mesh_pos[:, 1] + 0.0
    zflat = jnp.zeros((PT * 8,), f32)

    tbl = _dedup_scatter(key3)
    featf, agga, aggb = _dedup_features(key3, lo2, hi2, mx, my, zflat, tbl)
    feat = featf.reshape(EH, 8)
    aggd = jnp.stack([agga.reshape(16, NSC, GC), aggb.reshape(16, NSC, GC)])
    lohi = jnp.stack([lo2.reshape(NT, NGC // 2, GC), hi2.reshape(NT, NGC // 2, GC)])
    zer = jnp.zeros((TR, 128), f32)

    nt1h = jax.nn.one_hot(node_type[:, 0], 9, dtype=f32)
    nf16 = jnp.concatenate([velocity, nt1h, jnp.zeros((N, 5), f32)], axis=1)

    def mlp3(p):
        (w1, b1), (w2, b2), (w3, b3) = p
        return w1, _row(b1), w2, _row(b2), w3, _row(b3)

    nw1, nb1, nw2, nb2, nw3, nb3 = mlp3(params['enc_node']['mlp'])
    nw1p = jnp.zeros((16, 128), f32).at[:11].set(nw1)
    ng, nbb = params['enc_node']['ln']
    nmean, nstd = _node_stats(nf16)
    nodes = _node_encoder(nf16, nmean, nstd, nw1p, nb1, nw2, nb2, nw3, nb3,
                          _row(ng), _row(nbb))

    ew1, eb1, ew2, eb2, ew3, eb3 = mlp3(params['enc_edge']['mlp'])
    ew1p = jnp.zeros((8, 128), f32).at[:3].set(ew1)
    eg, ebb = params['enc_edge']['ln']
    esums = _edge_sums(feat)
    edges = _edge_encoder(feat, esums, ew1p, eb1, ew2, eb2, ew3, eb3,
                          _row(eg), _row(ebb))

    for blk in params['blocks']:
        (w1, b1), (w2, b2), (w3, b3) = blk['edge_mlp']
        eg_, ebb_ = blk['edge_ln']
        (nw1_, nb1_), (nw2_, nb2_), (nw3_, nb3_) = blk['node_mlp']
        ng_, nbb_ = blk['node_ln']

        g2 = _gather_nodes(nodes, lohi)
        edges = _edge_mlp(edges, g2, w1, _row(b1),
                          w2, _row(b2), w3, _row(b3), _row(eg_), _row(ebb_))
        p = _segment_sum(edges, aggd, zer)
        nodes = _node_mlp(nodes, p, nw1_, _row(nb1_),
                          nw2_, _row(nb2_), nw3_, _row(nb3_),
                          _row(ng_), _row(nbb_))

    (dw1, db1), (dw2, db2), (dw3, db3) = params['dec']['mlp']
    dw3p = jnp.zeros((128, 128), f32).at[:, :2].set(dw3)
    db3p = jnp.zeros((1, 128), f32).at[0, :2].set(db3)
    out = _decoder(nodes, dw1, _row(db1), dw2, _row(db2), dw3p, db3p)
    return out[:, :2]
